# baseline (speedup 1.0000x reference)
import jax
import jax.numpy as jnp
from jax import lax
from jax.experimental import pallas as pl
from jax.experimental.pallas import tpu as pltpu
from jax.experimental.pallas import tpu_sc as plsc

N = 10000
D = 128
NC = 2
NS = 16
NW = NC * NS
K = 128
NPASS = 10
N_TAB = 10240
RBLK = 1000


def _fill(buf, val):
    v16 = jnp.full((16,), val, jnp.float32)

    def row(r, c):
        for i in range(D // 16):
            buf[r, pl.ds(i * 16, 16)] = v16
        return c
    lax.fori_loop(0, K, row, 0)


def _zero_table(sid, sh, buf):
    nz = N_TAB // NS // K

    def zch(t, c):
        r0 = pl.multiple_of((sid * nz + t) * K, K)
        pltpu.sync_copy(buf, sh.at[pl.ds(r0, K)])
        return c
    lax.fori_loop(0, nz, zch, 0)


def _writeout(cid, sid, sh, out):
    rw = N_TAB // NS
    r0 = pl.multiple_of(sid * rw, 8)
    pltpu.sync_copy(sh.at[pl.ds(r0, rw)], out.at[cid, pl.ds(r0, rw)])


def _agg_body(h_hbm, src_hbm, dst_hbm, out, agg_sh, src_v, dst_v, rows_a,
              rows_b, sem_a, sem_b, ssem_a, ssem_b):
    cid = lax.axis_index("c")
    sid = lax.axis_index("s")
    wid = cid * NS + sid
    chp = src_v.shape[0]

    _fill(rows_a, 0.0)
    _zero_table(sid, agg_sh, rows_a)
    plsc.subcore_barrier()

    bufs = (rows_a, rows_b)
    gsems = (sem_a, sem_b)
    ssems = (ssem_a, ssem_b)

    def swait(q):
        pltpu.make_async_copy(bufs[q], agg_sh.at[pl.ds(0, K)],
                              ssems[q]).wait()

    for p in range(NPASS):
        off = p * chp
        pltpu.sync_copy(src_hbm.at[wid, pl.ds(off, chp)], src_v)
        pltpu.sync_copy(dst_hbm.at[wid, pl.ds(off, chp)], dst_v)

        pltpu.async_copy(h_hbm.at[src_v.at[0]], rows_a, sem_a)
        pltpu.async_copy(h_hbm.at[src_v.at[1]], rows_b, sem_b)

        def step(t, c):
            j0 = 2 * t
            j1 = 2 * t + 1
            pltpu.make_async_copy(h_hbm.at[src_v.at[j0]], rows_a,
                                  sem_a).wait()
            pltpu.async_copy(rows_a, agg_sh.at[dst_v.at[j0]], ssem_a,
                             add=True)
            pltpu.make_async_copy(h_hbm.at[src_v.at[j1]], rows_b,
                                  sem_b).wait()
            pltpu.async_copy(rows_b, agg_sh.at[dst_v.at[j1]], ssem_b,
                             add=True)
            swait(0)

            @pl.when(j0 + 2 < chp)
            def _():
                pltpu.async_copy(h_hbm.at[src_v.at[j0 + 2]], rows_a, sem_a)
            swait(1)

            @pl.when(j1 + 2 < chp)
            def _():
                pltpu.async_copy(h_hbm.at[src_v.at[j1 + 2]], rows_b, sem_b)
            return c
        lax.fori_loop(0, chp // 2, step, 0)
    plsc.subcore_barrier()
    _writeout(cid, sid, agg_sh, out)


def _deg_body(dst_hbm, out, deg_sh, dst_v, buf):
    cid = lax.axis_index("c")
    sid = lax.axis_index("s")
    wid = cid * NS + sid
    chp = dst_v.shape[0]

    _fill(buf, 0.0)
    _zero_table(sid, deg_sh, buf)
    _fill(buf, 1.0)
    plsc.subcore_barrier()

    for p in range(NPASS):
        off = p * chp
        pltpu.sync_copy(dst_hbm.at[wid, pl.ds(off, chp)], dst_v)

        def step(j, c):
            pltpu.sync_copy(buf, deg_sh.at[dst_v.at[j]], add=True)
            return c
        lax.fori_loop(0, chp, step, 0)
    plsc.subcore_barrier()
    _writeout(cid, sid, deg_sh, out)


def _sc_mesh():
    return plsc.VectorSubcoreMesh(core_axis_name="c", subcore_axis_name="s",
                                  num_cores=NC, num_subcores=NS)


def _make_sc_agg(chp):
    assert chp % 2 == 0
    out_type = [jax.ShapeDtypeStruct((NC, N_TAB, D), jnp.float32)]
    scratch = [pltpu.VMEM_SHARED((N_TAB, D), jnp.float32),
               pltpu.VMEM((chp, K), jnp.int32),
               pltpu.VMEM((chp, K), jnp.int32),
               pltpu.VMEM((K, D), jnp.float32),
               pltpu.VMEM((K, D), jnp.float32),
               pltpu.SemaphoreType.DMA,
               pltpu.SemaphoreType.DMA,
               pltpu.SemaphoreType.DMA,
               pltpu.SemaphoreType.DMA]
    return pl.kernel(_agg_body, out_type=out_type, mesh=_sc_mesh(),
                     scratch_types=scratch)


def _make_sc_deg(chp):
    out_type = [jax.ShapeDtypeStruct((NC, N_TAB, D), jnp.float32)]
    scratch = [pltpu.VMEM_SHARED((N_TAB, D), jnp.float32),
               pltpu.VMEM((chp, K), jnp.int32),
               pltpu.VMEM((K, D), jnp.float32)]
    return pl.kernel(_deg_body, out_type=out_type, mesh=_sc_mesh(),
                     scratch_types=scratch)


def _tc_layer(h, pa, dg, wr, wn, b2d, act):
    def body(h_ref, p0_ref, p1_ref, d0_ref, d1_ref, wr_ref, wn_ref, b_ref,
             o_ref):
        dsum = d0_ref[0] + d1_ref[0]
        rdeg = 1.0 / jnp.maximum(dsum, 1.0)
        mean = (p0_ref[0] + p1_ref[0]) * rdeg
        acc = jnp.dot(h_ref[...], wr_ref[...],
                      preferred_element_type=jnp.float32)
        acc = acc + jnp.dot(mean, wn_ref[...],
                            preferred_element_type=jnp.float32)
        acc = acc + b_ref[...]
        if act == "relu":
            acc = jnp.maximum(acc, 0.0)
        elif act == "lsm":
            m = jnp.max(acc, axis=1, keepdims=True)
            e = jnp.exp(acc - m)
            s = jnp.sum(e, axis=1, keepdims=True)
            acc = acc - m - jnp.log(s)
        o_ref[...] = acc

    row = pl.BlockSpec((RBLK, D), lambda i: (i, 0))
    pc0 = pl.BlockSpec((1, RBLK, D), lambda i: (0, i, 0))
    pc1 = pl.BlockSpec((1, RBLK, D), lambda i: (1, i, 0))
    wsp = pl.BlockSpec((D, D), lambda i: (0, 0))
    bsp = pl.BlockSpec((1, D), lambda i: (0, 0))
    return pl.pallas_call(
        body, grid=(N // RBLK,),
        in_specs=[row, pc0, pc1, pc0, pc1, wsp, wsp, bsp],
        out_specs=row,
        out_shape=jax.ShapeDtypeStruct((N, D), jnp.float32),
    )(h, pa, pa, dg, dg, wr, wn, b2d)


def kernel(x, local_edges_list, remote_edges_list,
           W_root0, W_neigh0, b0,
           W_root1, W_neigh1, b1,
           W_root2, W_neigh2, b2):
    src = local_edges_list[0]
    dst = local_edges_list[1]
    e = src.shape[0]
    chp = -(-e // (NW * NPASS * K))
    chp = -(-chp // 8) * 8
    e_pad = NW * NPASS * chp * K
    pad = e_pad - e
    src_pad = (jnp.arange(pad, dtype=jnp.int32) * 61) % N
    src3 = jnp.concatenate([src, src_pad]).reshape(NW, NPASS * chp, K)
    dst3 = jnp.concatenate(
        [dst, jnp.full((pad,), N, jnp.int32)]).reshape(NW, NPASS * chp, K)

    agg = _make_sc_agg(chp)
    deg = _make_sc_deg(chp)

    b0r = b0.reshape(1, D)
    b1r = b1.reshape(1, D)
    b2r = b2.reshape(1, D)

    dg, = deg(dst3)
    pa, = agg(x, src3, dst3)
    h1 = _tc_layer(x, pa, dg, W_root0, W_neigh0, b0r, "relu")
    pa, = agg(h1, src3, dst3)
    h2 = _tc_layer(h1, pa, dg, W_root1, W_neigh1, b1r, "relu")
    pa, = agg(h2, src3, dst3)
    return _tc_layer(h2, pa, dg, W_root2, W_neigh2, b2r, "lsm")


# speedup vs baseline: 1.1542x; 1.1542x over previous
import jax
import jax.numpy as jnp
from jax import lax
from jax.experimental import pallas as pl
from jax.experimental.pallas import tpu as pltpu
from jax.experimental.pallas import tpu_sc as plsc

N = 10000
D = 128
NC = 2
NS = 16
NW = NC * NS
K = 128
NPASS = 10
N_TAB = 10240
RBLK = 1000


def _fill(buf, val):
    v16 = jnp.full((16,), val, jnp.float32)

    def row(r, c):
        for i in range(D // 16):
            buf[r, pl.ds(i * 16, 16)] = v16
        return c
    lax.fori_loop(0, K, row, 0)


def _zero_table(sid, sh, buf):
    nz = N_TAB // NS // K

    def zch(t, c):
        r0 = pl.multiple_of((sid * nz + t) * K, K)
        pltpu.sync_copy(buf, sh.at[pl.ds(r0, K)])
        return c
    lax.fori_loop(0, nz, zch, 0)


def _writeout(cid, sid, sh, out):
    rw = N_TAB // NS
    r0 = pl.multiple_of(sid * rw, 8)
    pltpu.sync_copy(sh.at[pl.ds(r0, rw)], out.at[cid, pl.ds(r0, rw)])


def _agg_body(h_hbm, src_hbm, dst_hbm, out, agg_sh, src_v, dst_v, rows_a,
              rows_b, sem_a, sem_b):
    cid = lax.axis_index("c")
    sid = lax.axis_index("s")
    wid = cid * NS + sid
    chp = src_v.shape[0]

    _fill(rows_a, 0.0)
    _zero_table(sid, agg_sh, rows_a)
    plsc.subcore_barrier()

    bufs = (rows_a, rows_b)
    sems = (sem_a, sem_b)

    for p in range(NPASS):
        off = p * chp
        pltpu.sync_copy(src_hbm.at[wid, pl.ds(off, chp)], src_v)
        pltpu.sync_copy(dst_hbm.at[wid, pl.ds(off, chp)], dst_v)

        pltpu.async_copy(h_hbm.at[src_v.at[0]], rows_a, sem_a)
        pltpu.async_copy(h_hbm.at[src_v.at[1]], rows_b, sem_b)

        def step(t, c):
            for q in range(2):
                j = 2 * t + q
                buf, sem = bufs[q], sems[q]
                pltpu.make_async_copy(h_hbm.at[src_v.at[j]], buf, sem).wait()
                pltpu.sync_copy(buf, agg_sh.at[dst_v.at[j]], add=True)

                @pl.when(j + 2 < chp)
                def _():
                    pltpu.async_copy(h_hbm.at[src_v.at[j + 2]], buf, sem)
            return c
        lax.fori_loop(0, chp // 2, step, 0)
    plsc.subcore_barrier()
    _writeout(cid, sid, agg_sh, out)


def _deg_body(dst_hbm, out, deg_sh, dst_v, buf):
    cid = lax.axis_index("c")
    sid = lax.axis_index("s")
    wid = cid * NS + sid
    chp = dst_v.shape[0]

    _fill(buf, 0.0)
    _zero_table(sid, deg_sh, buf)
    _fill(buf, 1.0)
    plsc.subcore_barrier()

    for p in range(NPASS):
        off = p * chp
        pltpu.sync_copy(dst_hbm.at[wid, pl.ds(off, chp)], dst_v)

        def step(j, c):
            pltpu.sync_copy(buf, deg_sh.at[dst_v.at[j]], add=True)
            return c
        lax.fori_loop(0, chp, step, 0)
    plsc.subcore_barrier()
    _writeout(cid, sid, deg_sh, out)


def _sc_mesh():
    return plsc.VectorSubcoreMesh(core_axis_name="c", subcore_axis_name="s",
                                  num_cores=NC, num_subcores=NS)


def _make_sc_agg(chp):
    assert chp % 2 == 0
    out_type = [jax.ShapeDtypeStruct((NC, N_TAB, D), jnp.float32)]
    scratch = [pltpu.VMEM_SHARED((N_TAB, D), jnp.float32),
               pltpu.VMEM((chp, K), jnp.int32),
               pltpu.VMEM((chp, K), jnp.int32),
               pltpu.VMEM((K, D), jnp.float32),
               pltpu.VMEM((K, D), jnp.float32),
               pltpu.SemaphoreType.DMA,
               pltpu.SemaphoreType.DMA]
    return pl.kernel(_agg_body, out_type=out_type, mesh=_sc_mesh(),
                     scratch_types=scratch)


def _make_sc_deg(chp):
    out_type = [jax.ShapeDtypeStruct((NC, N_TAB, D), jnp.float32)]
    scratch = [pltpu.VMEM_SHARED((N_TAB, D), jnp.float32),
               pltpu.VMEM((chp, K), jnp.int32),
               pltpu.VMEM((K, D), jnp.float32)]
    return pl.kernel(_deg_body, out_type=out_type, mesh=_sc_mesh(),
                     scratch_types=scratch)


def _tc_layer(h, pa, dg, wr, wn, b2d, act):
    def body(h_ref, p0_ref, p1_ref, d0_ref, d1_ref, wr_ref, wn_ref, b_ref,
             o_ref):
        dsum = d0_ref[0] + d1_ref[0]
        rdeg = 1.0 / jnp.maximum(dsum, 1.0)
        mean = (p0_ref[0] + p1_ref[0]) * rdeg
        acc = jnp.dot(h_ref[...], wr_ref[...],
                      preferred_element_type=jnp.float32)
        acc = acc + jnp.dot(mean, wn_ref[...],
                            preferred_element_type=jnp.float32)
        acc = acc + b_ref[...]
        if act == "relu":
            acc = jnp.maximum(acc, 0.0)
        elif act == "lsm":
            m = jnp.max(acc, axis=1, keepdims=True)
            e = jnp.exp(acc - m)
            s = jnp.sum(e, axis=1, keepdims=True)
            acc = acc - m - jnp.log(s)
        o_ref[...] = acc

    row = pl.BlockSpec((RBLK, D), lambda i: (i, 0))
    pc0 = pl.BlockSpec((1, RBLK, D), lambda i: (0, i, 0))
    pc1 = pl.BlockSpec((1, RBLK, D), lambda i: (1, i, 0))
    wsp = pl.BlockSpec((D, D), lambda i: (0, 0))
    bsp = pl.BlockSpec((1, D), lambda i: (0, 0))
    return pl.pallas_call(
        body, grid=(N // RBLK,),
        in_specs=[row, pc0, pc1, pc0, pc1, wsp, wsp, bsp],
        out_specs=row,
        out_shape=jax.ShapeDtypeStruct((N, D), jnp.float32),
    )(h, pa, pa, dg, dg, wr, wn, b2d)


def kernel(x, local_edges_list, remote_edges_list,
           W_root0, W_neigh0, b0,
           W_root1, W_neigh1, b1,
           W_root2, W_neigh2, b2):
    src = local_edges_list[0]
    dst = local_edges_list[1]
    e = src.shape[0]
    chp = -(-e // (NW * NPASS * K))
    chp = -(-chp // 8) * 8
    e_pad = NW * NPASS * chp * K
    pad = e_pad - e
    src_pad = (jnp.arange(pad, dtype=jnp.int32) * 61) % N
    src3 = jnp.concatenate([src, src_pad]).reshape(NW, NPASS * chp, K)
    dst3 = jnp.concatenate(
        [dst, jnp.full((pad,), N, jnp.int32)]).reshape(NW, NPASS * chp, K)

    agg = _make_sc_agg(chp)
    deg = _make_sc_deg(chp)

    b0r = b0.reshape(1, D)
    b1r = b1.reshape(1, D)
    b2r = b2.reshape(1, D)

    dg, = deg(dst3)
    pa, = agg(x, src3, dst3)
    h1 = _tc_layer(x, pa, dg, W_root0, W_neigh0, b0r, "relu")
    pa, = agg(h1, src3, dst3)
    h2 = _tc_layer(h1, pa, dg, W_root1, W_neigh1, b1r, "relu")
    pa, = agg(h2, src3, dst3)
    return _tc_layer(h2, pa, dg, W_root2, W_neigh2, b2r, "lsm")
